# baseline (speedup 1.0000x reference)
import jax
import jax.numpy as jnp
import numpy as np
from jax import lax
from jax.experimental import pallas as pl
from jax.experimental.pallas import tpu as pltpu
from jax.experimental.pallas import tpu_sc as plsc

_NUM_CLASSES = 1000
_SMOOTH = 0.1
_B = 16384
_L = 50

_HIT = np.float32(np.float32(1.0) * np.float32(1.0 - _SMOOTH)
                  + np.float32(_SMOOTH / _NUM_CLASSES))
_BG = np.float32(_SMOOTH / _NUM_CLASSES)

_NC = 2
_NS = 16
_NW = _NC * _NS
_ROWS_PER_W = _B // _NW
_CHUNK = 32
_NCHUNK = _ROWS_PER_W // _CHUNK
_COLS = (0, 16, 32, _L - 16)


def _body(x_hbm, out_hbm, idx_v0, idx_v1, buf_v0, buf_v1, sem0, sem1):
    cid = lax.axis_index("c")
    sid = lax.axis_index("s")
    wid = sid * _NC + cid

    idx_refs = (idx_v0, idx_v1)
    buf_refs = (buf_v0, buf_v1)
    sems = (sem0, sem1)

    hit = jnp.full((16,), _HIT, dtype=jnp.float32)
    bg = jnp.full((16,), _BG, dtype=jnp.float32)

    for b in range(2):
        buf = buf_refs[b]

        def init_step(r, _, buf=buf):
            def col_step(c, __):
                buf[r, pl.ds(c * 16, 16)] = bg
                return 0
            lax.fori_loop(0, _NUM_CLASSES // 16, col_step, 0)
            buf[r, pl.ds(_NUM_CLASSES - 16, 16)] = bg
            return 0
        lax.fori_loop(0, _CHUNK, init_step, 0)

    def scatter_pass(idx_ref, buf_ref, value_vec):
        def row_step(r, _):
            row_vec = jnp.full((16,), r, dtype=jnp.int32)
            for c in _COLS:
                cls = idx_ref[r, pl.ds(c, 16)]
                plsc.store_scatter(buf_ref, [row_vec, cls], value_vec)
            return 0
        lax.fori_loop(0, _CHUNK, row_step, 0)

    for t in range(_NCHUNK):
        s = t % 2
        idx_ref, buf_ref, sem = idx_refs[s], buf_refs[s], sems[s]
        row_base = wid * _ROWS_PER_W + t * _CHUNK
        if t >= 2:
            pltpu.make_async_copy(
                buf_ref, out_hbm.at[pl.ds(row_base, _CHUNK)], sem).wait()
            scatter_pass(idx_ref, buf_ref, bg)
        pltpu.sync_copy(x_hbm.at[pl.ds(row_base, _CHUNK)], idx_ref)
        scatter_pass(idx_ref, buf_ref, hit)
        pltpu.async_copy(buf_ref, out_hbm.at[pl.ds(row_base, _CHUNK)], sem)

    for t in (_NCHUNK - 2, _NCHUNK - 1):
        s = t % 2
        row_base = wid * _ROWS_PER_W + t * _CHUNK
        pltpu.make_async_copy(
            buf_refs[s], out_hbm.at[pl.ds(row_base, _CHUNK)], sems[s]).wait()


@jax.jit
def _multihot(x):
    mesh = plsc.VectorSubcoreMesh(core_axis_name="c", subcore_axis_name="s")
    fn = pl.kernel(
        _body,
        out_type=jax.ShapeDtypeStruct((_B, _NUM_CLASSES), jnp.float32),
        mesh=mesh,
        scratch_types=[
            pltpu.VMEM((_CHUNK, _L), jnp.int32),
            pltpu.VMEM((_CHUNK, _L), jnp.int32),
            pltpu.VMEM((_CHUNK, _NUM_CLASSES), jnp.float32),
            pltpu.VMEM((_CHUNK, _NUM_CLASSES), jnp.float32),
            pltpu.SemaphoreType.DMA,
            pltpu.SemaphoreType.DMA,
        ],
        compiler_params=pltpu.CompilerParams(needs_layout_passes=False,
                                             use_tc_tiling_on_sc=True),
    )
    return fn(x)


def kernel(x):
    return _multihot(x.astype(jnp.int32))


# speedup vs baseline: 1.7778x; 1.7778x over previous
import jax
import jax.numpy as jnp
import numpy as np
from jax import lax
from jax.experimental import pallas as pl
from jax.experimental.pallas import tpu as pltpu
from jax.experimental.pallas import tpu_sc as plsc

_NUM_CLASSES = 1000
_SMOOTH = 0.1
_B = 16384
_L = 50

_HIT = np.float32(np.float32(1.0) * np.float32(1.0 - _SMOOTH)
                  + np.float32(_SMOOTH / _NUM_CLASSES))
_BG = np.float32(_SMOOTH / _NUM_CLASSES)

_NC = 2
_NS = 16
_NW = _NC * _NS
_BATCH_PER_W = _B // _NW
_BLK = 128
_NBLK = _BATCH_PER_W // _BLK
_PIECES = ((0, 16), (16, 16), (32, 16), (48, 2))


def _body(x_hbm, out_hbm, idx_v, idxt_v, buf_v):
    cid = lax.axis_index("c")
    sid = lax.axis_index("s")
    wid = sid * _NC + cid

    hit = jnp.full((16,), _HIT, dtype=jnp.float32)
    bg = jnp.full((16,), _BG, dtype=jnp.float32)
    lanes = lax.iota(jnp.int32, 16)

    def init_row(c, _):
        def init_col(g, __):
            buf_v[c, pl.ds(g * 16, 16)] = bg
            return 0
        lax.fori_loop(0, _BLK // 16, init_col, 0)
        return 0
    lax.fori_loop(0, _NUM_CLASSES, init_row, 0)

    def scatter_block(bbase, value_vec):
        for (l0, nrows) in _PIECES:
            piece = idxt_v if nrows == 2 else idx_v
            pltpu.sync_copy(x_hbm.at[pl.ds(l0, nrows), pl.ds(bbase, _BLK)],
                            piece)

            def row_step(l, _, piece=piece):
                def col_step(g, __):
                    b_loc = jnp.full((16,), g * 16, jnp.int32) + lanes
                    cls = piece[l, pl.ds(g * 16, 16)]
                    plsc.store_scatter(buf_v, [cls, b_loc], value_vec)
                    return 0
                lax.fori_loop(0, _BLK // 16, col_step, 0)
                return 0
            lax.fori_loop(0, nrows, row_step, 0)

    for t in range(_NBLK):
        bbase = wid * _BATCH_PER_W + t * _BLK
        if t > 0:
            scatter_block(bbase - _BLK, bg)
        scatter_block(bbase, hit)
        pltpu.sync_copy(buf_v, out_hbm.at[:, pl.ds(bbase, _BLK)])


@jax.jit
def _multihot_t(x_t):
    mesh = plsc.VectorSubcoreMesh(core_axis_name="c", subcore_axis_name="s")
    fn = pl.kernel(
        _body,
        out_type=jax.ShapeDtypeStruct((_NUM_CLASSES, _B), jnp.float32),
        mesh=mesh,
        scratch_types=[
            pltpu.VMEM((16, _BLK), jnp.int32),
            pltpu.VMEM((2, _BLK), jnp.int32),
            pltpu.VMEM((_NUM_CLASSES, _BLK), jnp.float32),
        ],
        compiler_params=pltpu.CompilerParams(needs_layout_passes=False),
    )
    return fn(x_t)


def kernel(x):
    out_t = _multihot_t(x.astype(jnp.int32).T)
    return out_t.T


# speedup vs baseline: 1.8015x; 1.0134x over previous
import jax
import jax.numpy as jnp
import numpy as np
from jax import lax
from jax.experimental import pallas as pl
from jax.experimental.pallas import tpu as pltpu
from jax.experimental.pallas import tpu_sc as plsc

_NUM_CLASSES = 1000
_SMOOTH = 0.1
_B = 16384
_L = 50

_HIT = np.float32(np.float32(1.0) * np.float32(1.0 - _SMOOTH)
                  + np.float32(_SMOOTH / _NUM_CLASSES))
_BG = np.float32(_SMOOTH / _NUM_CLASSES)

_NC = 2
_NS = 16
_NW = _NC * _NS
_BATCH_PER_W = _B // _NW
_BLK = 128
_NBLK = _BATCH_PER_W // _BLK
_PIECES = ((0, 16), (16, 16), (32, 16), (48, 2))


def _body(x_hbm, out_hbm, idx_v, idxt_v, buf_v):
    cid = lax.axis_index("c")
    sid = lax.axis_index("s")
    wid = sid * _NC + cid

    hit = jnp.full((16,), _HIT, dtype=jnp.float32)
    bg = jnp.full((16,), _BG, dtype=jnp.float32)
    lanes = lax.iota(jnp.int32, 16)
    b_locs = [jnp.full((16,), g * 16, jnp.int32) + lanes
              for g in range(_BLK // 16)]

    def init_row(c, _):
        def init_col(g, __):
            buf_v[c, pl.ds(g * 16, 16)] = bg
            return 0
        lax.fori_loop(0, _BLK // 16, init_col, 0)
        return 0
    lax.fori_loop(0, _NUM_CLASSES, init_row, 0)

    def scatter_block(bbase, value_vec):
        for (l0, nrows) in _PIECES:
            piece = idxt_v if nrows == 2 else idx_v
            pltpu.sync_copy(x_hbm.at[pl.ds(l0, nrows), pl.ds(bbase, _BLK)],
                            piece)

            def row_step(l, _, piece=piece):
                for g in range(_BLK // 16):
                    cls = piece[l, pl.ds(g * 16, 16)]
                    plsc.store_scatter(buf_v, [cls, b_locs[g]], value_vec)
                return 0
            lax.fori_loop(0, nrows, row_step, 0)

    for t in range(_NBLK):
        bbase = wid * _BATCH_PER_W + t * _BLK
        if t > 0:
            scatter_block(bbase - _BLK, bg)
        scatter_block(bbase, hit)
        pltpu.sync_copy(buf_v, out_hbm.at[:, pl.ds(bbase, _BLK)])


@jax.jit
def _multihot_t(x_t):
    mesh = plsc.VectorSubcoreMesh(core_axis_name="c", subcore_axis_name="s")
    fn = pl.kernel(
        _body,
        out_type=jax.ShapeDtypeStruct((_NUM_CLASSES, _B), jnp.float32),
        mesh=mesh,
        scratch_types=[
            pltpu.VMEM((16, _BLK), jnp.int32),
            pltpu.VMEM((2, _BLK), jnp.int32),
            pltpu.VMEM((_NUM_CLASSES, _BLK), jnp.float32),
        ],
        compiler_params=pltpu.CompilerParams(needs_layout_passes=False),
    )
    return fn(x_t)


def kernel(x):
    out_t = _multihot_t(x.astype(jnp.int32).T)
    return out_t.T


# speedup vs baseline: 2.1586x; 1.1982x over previous
import jax
import jax.numpy as jnp
import numpy as np
from jax import lax
from jax.experimental import pallas as pl
from jax.experimental.pallas import tpu as pltpu
from jax.experimental.pallas import tpu_sc as plsc

_NUM_CLASSES = 1000
_SMOOTH = 0.1
_B = 16384
_L = 50

_HIT = np.float32(np.float32(1.0) * np.float32(1.0 - _SMOOTH)
                  + np.float32(_SMOOTH / _NUM_CLASSES))
_BG = np.float32(_SMOOTH / _NUM_CLASSES)

_NC = 2
_NS = 16
_NW = _NC * _NS
_BATCH_PER_W = _B // _NW
_BLK = 128
_NBLK = _BATCH_PER_W // _BLK
_PIECES = ((0, 16), (16, 16), (32, 16), (48, 2))


def _body(x_hbm, out_hbm, idx_v, idxt_v, buf_v):
    cid = lax.axis_index("c")
    sid = lax.axis_index("s")
    wid = sid * _NC + cid

    hit = jnp.full((16,), _HIT, dtype=jnp.float32)
    bg = jnp.full((16,), _BG, dtype=jnp.float32)
    lanes = lax.iota(jnp.int32, 16)
    b_locs = [jnp.full((16,), g * 16, jnp.int32) + lanes
              for g in range(_BLK // 16)]

    def init_row(c, _):
        def init_col(g, __):
            buf_v[c, pl.ds(g * 16, 16)] = bg
            return 0
        lax.fori_loop(0, _BLK // 16, init_col, 0)
        return 0
    lax.fori_loop(0, _NUM_CLASSES, init_row, 0)

    def scatter_block(bbase, value_vec):
        for (l0, nrows) in _PIECES:
            piece = idxt_v if nrows == 2 else idx_v
            pltpu.sync_copy(x_hbm.at[pl.ds(l0, nrows), pl.ds(bbase, _BLK)],
                            piece)

            def row_step(l, _, piece=piece):
                clss = [piece[l, pl.ds(g * 16, 16)]
                        for g in range(_BLK // 16)]
                for g in range(_BLK // 16):
                    plsc.store_scatter(buf_v, [clss[g], b_locs[g]],
                                       value_vec)
                return 0
            lax.fori_loop(0, nrows, row_step, 0)

    for t in range(_NBLK):
        bbase = wid * _BATCH_PER_W + t * _BLK
        if t > 0:
            scatter_block(bbase - _BLK, bg)
        scatter_block(bbase, hit)
        pltpu.sync_copy(buf_v, out_hbm.at[:, pl.ds(bbase, _BLK)])


@jax.jit
def _multihot_t(x_t):
    mesh = plsc.VectorSubcoreMesh(core_axis_name="c", subcore_axis_name="s")
    fn = pl.kernel(
        _body,
        out_type=jax.ShapeDtypeStruct((_NUM_CLASSES, _B), jnp.float32),
        mesh=mesh,
        scratch_types=[
            pltpu.VMEM((16, _BLK), jnp.int32),
            pltpu.VMEM((2, _BLK), jnp.int32),
            pltpu.VMEM((_NUM_CLASSES, _BLK), jnp.float32),
        ],
        compiler_params=pltpu.CompilerParams(needs_layout_passes=False),
    )
    return fn(x_t)


def kernel(x):
    out_t = _multihot_t(x.astype(jnp.int32).T)
    return out_t.T
